# baseline (speedup 1.0000x reference)
import functools

import jax
import jax.numpy as jnp
from jax import lax
from jax.experimental import pallas as pl
from jax.experimental.pallas import tpu as pltpu
from jax.experimental.pallas import tpu_sc as plsc

N = 10000
E = 320000
D = 128
H = 128
OUT = 64
L = 4

NC = 2
NS = 16
NW = NC * NS
EB = 128
CH = 10
NB = -(-E // (NW * EB * CH)) * CH
E_PAD = NW * NB * EB
SH = (-(-(N + 16) // NS) + 7) // 8 * 8
NP = NS * SH
KB = 2

@functools.lru_cache(maxsize=1)
def _make_sc_agg():
    mesh = plsc.VectorSubcoreMesh(
        core_axis_name="c", subcore_axis_name="s",
        num_cores=NC, num_subcores=NS)

    @functools.partial(
        pl.kernel,
        out_type=jax.ShapeDtypeStruct((NC, NP, D), jnp.float32),
        mesh=mesh,
        scratch_types=[
            pltpu.VMEM((CH, EB), jnp.int32),
            pltpu.VMEM((CH, EB), jnp.int32),
            [pltpu.VMEM((EB, D), jnp.float32) for _ in range(KB)],
            pltpu.VMEM_SHARED((NP, D), jnp.float32),
            [pltpu.SemaphoreType.DMA for _ in range(KB)],
        ],
    )
    def sc_agg(h_hbm, src_hbm, dst_hbm, z_hbm, out_hbm,
               src_v, dst_v, bufs, acc_sh, sems):
        cid = lax.axis_index("c")
        sid = lax.axis_index("s")
        wid = sid * NC + cid
        pltpu.sync_copy(z_hbm, acc_sh.at[pl.ds(sid * SH, SH)])
        plsc.subcore_barrier()

        def body(t, carry):
            pltpu.sync_copy(src_hbm.at[wid, t], src_v)
            pltpu.sync_copy(dst_hbm.at[wid, t], dst_v)
            pltpu.async_copy(h_hbm.at[src_v.at[0]], bufs[0], sems[0])
            for i in range(CH):
                pltpu.make_async_copy(
                    h_hbm.at[pl.ds(0, EB)], bufs[i % KB],
                    sems[i % KB]).wait()
                if i + 1 < CH:
                    pltpu.async_copy(h_hbm.at[src_v.at[i + 1]],
                                     bufs[(i + 1) % KB], sems[(i + 1) % KB])
                pltpu.sync_copy(bufs[i % KB], acc_sh.at[dst_v.at[i]],
                                add=True)
            return carry

        lax.fori_loop(0, NB // CH, body, 0)
        plsc.subcore_barrier()
        pltpu.sync_copy(acc_sh.at[pl.ds(sid * SH, SH)],
                        out_hbm.at[cid, pl.ds(sid * SH, SH)])

    return sc_agg


def _sc_agg(h, src_r, dst_r, zeros):
    return _make_sc_agg()(h, src_r, dst_r, zeros)


R = 1000
G = N // R

_F32 = jnp.float32


def _score_init_body(h_ref, w_ref, b_ref, out_ref):
    out_ref[...] = (
        jnp.dot(h_ref[...], w_ref[...], preferred_element_type=jnp.float32)
        + b_ref[...])


def _score_init(h, w, b):
    return pl.pallas_call(
        _score_init_body,
        grid=(G,),
        in_specs=[pl.BlockSpec((R, D), lambda i: (i, 0)),
                  pl.BlockSpec((D, OUT), lambda i: (0, 0)),
                  pl.BlockSpec((1, OUT), lambda i: (0, 0))],
        out_specs=pl.BlockSpec((R, OUT), lambda i: (i, 0)),
        out_shape=jax.ShapeDtypeStruct((N, OUT), _F32),
    )(h, w, b)


def _bn(x, mu_row, sq_row, g_ref, bt_ref):
    mu = mu_row * (1.0 / N)
    var = sq_row * (1.0 / N) - mu * mu
    inv = g_ref[...] * lax.rsqrt(var + 1e-5)
    return (x - mu) * inv + bt_ref[...]


def _make_layer_body(outer_bn):
    def body(scale_ref, agg_ref, h_ref, w1_ref, b1_ref, g1_ref, bt1_ref,
             w2_ref, b2_ref, g2_ref, bt2_ref, pw_ref, pb_ref, score_ref,
             h_out_ref, score_out_ref, z1_scr, z2_scr, stats):
        p = pl.program_id(0)
        i = pl.program_id(1)
        rows = pl.ds(i * R, R)

        @pl.when(p == 0)
        def _():
            @pl.when(i == 0)
            def _():
                stats[...] = jnp.zeros_like(stats)
            pooled = agg_ref[0] + agg_ref[1] + scale_ref[...] * h_ref[...]
            z = (jnp.dot(pooled, w1_ref[...],
                         preferred_element_type=jnp.float32) + b1_ref[...])
            z1_scr[rows, :] = z
            stats[0:1, :] += jnp.sum(z, axis=0, keepdims=True)
            stats[1:2, :] += jnp.sum(z * z, axis=0, keepdims=True)

        @pl.when(p == 1)
        def _():
            z1 = z1_scr[rows, :]
            a = jnp.maximum(
                _bn(z1, stats[0:1, :], stats[1:2, :], g1_ref, bt1_ref), 0.0)
            z2 = (jnp.dot(a, w2_ref[...],
                          preferred_element_type=jnp.float32) + b2_ref[...])
            z2_scr[rows, :] = z2
            stats[2:3, :] += jnp.sum(z2, axis=0, keepdims=True)
            stats[3:4, :] += jnp.sum(z2 * z2, axis=0, keepdims=True)

        @pl.when(p == 2)
        def _():
            z2 = z2_scr[rows, :]
            if outer_bn:
                hnew = jnp.maximum(
                    _bn(z2, stats[2:3, :], stats[3:4, :], g2_ref, bt2_ref),
                    0.0)
            else:
                hnew = jnp.maximum(z2, 0.0)
            h_out_ref[...] = hnew
            score_out_ref[...] = (
                score_ref[...]
                + jnp.dot(hnew, pw_ref[...],
                          preferred_element_type=jnp.float32) + pb_ref[...])

    return body


def _full(shape):
    return pl.BlockSpec(shape, lambda p, i: (0, 0))


def _phase_rows(ph, w):
    return pl.BlockSpec(
        (R, w), lambda p, i, _ph=ph: (jnp.where(p == _ph, i, 0), 0))


_LAYER_SPECS = dict(
    grid=(3, G),
    in_specs=[
        _full((1, 1)),
        pl.BlockSpec((NC, R, D),
                     lambda p, i: (0, jnp.where(p == 0, i, 0), 0)),
        _phase_rows(0, D),
        _full((D, D)), _full((1, D)),
        _full((1, D)), _full((1, D)),
        _full((D, D)), _full((1, D)),
        _full((1, D)), _full((1, D)),
        _full((D, OUT)), _full((1, OUT)),
        _phase_rows(2, OUT),
    ],
    out_specs=(_phase_rows(2, D), _phase_rows(2, OUT)),
    out_shape=(jax.ShapeDtypeStruct((N, D), _F32),
               jax.ShapeDtypeStruct((N, OUT), _F32)),
    scratch_shapes=[pltpu.VMEM((N, D), _F32),
                    pltpu.VMEM((N, D), _F32),
                    pltpu.VMEM((8, D), _F32)],
)

def _layer_call(outer_bn):
    return pl.pallas_call(_make_layer_body(outer_bn), **_LAYER_SPECS)


def kernel(batch_features, batch_graphs, mlp_W1, mlp_b1, bn_in_gamma,
           bn_in_beta, mlp_W2, mlp_b2, outer_gamma, outer_beta, pred_W,
           pred_b, eps):
    src = batch_graphs[0]
    dst = batch_graphs[1]
    pad = E_PAD - E
    src_p = jnp.concatenate(
        [src, (jnp.arange(pad, dtype=jnp.int32) * 37) % N])
    dst_p = jnp.concatenate(
        [dst, N + (jnp.arange(pad, dtype=jnp.int32) % 96)])
    src_r = src_p.reshape(NW, NB // CH, CH, EB)
    dst_r = dst_p.reshape(NW, NB // CH, CH, EB)
    zeros = jnp.zeros((SH, D), _F32)

    h = batch_features
    scale = (1.0 + eps).reshape(L - 1, 1, 1)
    score = _score_init(h, pred_W[0], pred_b[0].reshape(1, OUT))
    unit = jnp.ones((1, H), _F32)
    zero_row = jnp.zeros((1, H), _F32)
    for i in range(L - 1):
        parts = _sc_agg(h, src_r, dst_r, zeros)
        if i == 0:
            g2, bt2 = unit, zero_row
            layer = _layer_call(False)
        else:
            g2 = outer_gamma[i - 1].reshape(1, H)
            bt2 = outer_beta[i - 1].reshape(1, H)
            layer = _layer_call(True)
        h, score = layer(
            scale[i], parts, h, mlp_W1[i], mlp_b1[i].reshape(1, H),
            bn_in_gamma[i].reshape(1, H), bn_in_beta[i].reshape(1, H),
            mlp_W2[i], mlp_b2[i].reshape(1, H), g2, bt2, pred_W[i + 1],
            pred_b[i + 1].reshape(1, OUT), score)
    return score


# speedup vs baseline: 1.0461x; 1.0461x over previous
import functools

import jax
import jax.numpy as jnp
from jax import lax
from jax.experimental import pallas as pl
from jax.experimental.pallas import tpu as pltpu
from jax.experimental.pallas import tpu_sc as plsc

N = 10000
E = 320000
D = 128
H = 128
OUT = 64
L = 4

NC = 2
NS = 16
NW = NC * NS
EB = 128
CH = 10
NB = -(-E // (NW * EB * CH)) * CH
E_PAD = NW * NB * EB
SH = (-(-(N + 16) // NS) + 7) // 8 * 8
NP = NS * SH
KB = 2

@functools.lru_cache(maxsize=1)
def _make_sc_agg():
    mesh = plsc.VectorSubcoreMesh(
        core_axis_name="c", subcore_axis_name="s",
        num_cores=NC, num_subcores=NS)

    @functools.partial(
        pl.kernel,
        out_type=jax.ShapeDtypeStruct((NC, NP, D), jnp.float32),
        mesh=mesh,
        scratch_types=[
            pltpu.VMEM((CH, EB), jnp.int32),
            pltpu.VMEM((CH, EB), jnp.int32),
            [pltpu.VMEM((EB, D), jnp.float32) for _ in range(KB)],
            pltpu.VMEM_SHARED((NP, D), jnp.float32),
            [pltpu.SemaphoreType.DMA for _ in range(KB)],
        ],
    )
    def sc_agg(h_hbm, src_hbm, dst_hbm, z_hbm, out_hbm,
               src_v, dst_v, bufs, acc_sh, sems):
        cid = lax.axis_index("c")
        sid = lax.axis_index("s")
        wid = sid * NC + cid
        pltpu.sync_copy(z_hbm, acc_sh.at[pl.ds(sid * SH, SH)])
        plsc.subcore_barrier()

        def body(t, carry):
            pltpu.sync_copy(src_hbm.at[wid, t], src_v)
            pltpu.sync_copy(dst_hbm.at[wid, t], dst_v)
            pltpu.async_copy(h_hbm.at[src_v.at[0]], bufs[0], sems[0])
            for i in range(CH):
                pltpu.make_async_copy(
                    h_hbm.at[pl.ds(0, EB)], bufs[i % KB],
                    sems[i % KB]).wait()
                if i + 1 < CH:
                    pltpu.async_copy(h_hbm.at[src_v.at[i + 1]],
                                     bufs[(i + 1) % KB], sems[(i + 1) % KB])
                pltpu.sync_copy(bufs[i % KB], acc_sh.at[dst_v.at[i]],
                                add=True)
            return carry

        lax.fori_loop(0, NB // CH, body, 0)
        plsc.subcore_barrier()
        pltpu.sync_copy(acc_sh.at[pl.ds(sid * SH, SH)],
                        out_hbm.at[cid, pl.ds(sid * SH, SH)])

    return sc_agg


def _sc_agg(h, src_r, dst_r, zeros):
    return _make_sc_agg()(h, src_r, dst_r, zeros)


R = 2000
G = N // R

_F32 = jnp.float32


def _score_init_body(h_ref, w_ref, b_ref, out_ref):
    out_ref[...] = (
        jnp.dot(h_ref[...], w_ref[...], preferred_element_type=jnp.float32)
        + b_ref[...])


def _score_init(h, w, b):
    return pl.pallas_call(
        _score_init_body,
        grid=(G,),
        in_specs=[pl.BlockSpec((R, D), lambda i: (i, 0)),
                  pl.BlockSpec((D, OUT), lambda i: (0, 0)),
                  pl.BlockSpec((1, OUT), lambda i: (0, 0))],
        out_specs=pl.BlockSpec((R, OUT), lambda i: (i, 0)),
        out_shape=jax.ShapeDtypeStruct((N, OUT), _F32),
    )(h, w, b)


def _bn(x, mu_row, sq_row, g_ref, bt_ref):
    mu = mu_row * (1.0 / N)
    var = sq_row * (1.0 / N) - mu * mu
    inv = g_ref[...] * lax.rsqrt(var + 1e-5)
    return (x - mu) * inv + bt_ref[...]


def _make_layer_body(outer_bn):
    def body(scale_ref, agg_ref, h_ref, w1_ref, b1_ref, g1_ref, bt1_ref,
             w2_ref, b2_ref, g2_ref, bt2_ref, pw_ref, pb_ref, score_ref,
             h_out_ref, score_out_ref, z1_scr, z2_scr, stats):
        p = pl.program_id(0)
        i = pl.program_id(1)
        rows = pl.ds(i * R, R)

        @pl.when(p == 0)
        def _():
            @pl.when(i == 0)
            def _():
                stats[...] = jnp.zeros_like(stats)
            pooled = agg_ref[0] + agg_ref[1] + scale_ref[...] * h_ref[...]
            z = (jnp.dot(pooled, w1_ref[...],
                         preferred_element_type=jnp.float32) + b1_ref[...])
            z1_scr[rows, :] = z
            stats[0:1, :] += jnp.sum(z, axis=0, keepdims=True)
            stats[1:2, :] += jnp.sum(z * z, axis=0, keepdims=True)

        @pl.when(p == 1)
        def _():
            z1 = z1_scr[rows, :]
            a = jnp.maximum(
                _bn(z1, stats[0:1, :], stats[1:2, :], g1_ref, bt1_ref), 0.0)
            z2 = (jnp.dot(a, w2_ref[...],
                          preferred_element_type=jnp.float32) + b2_ref[...])
            z2_scr[rows, :] = z2
            stats[2:3, :] += jnp.sum(z2, axis=0, keepdims=True)
            stats[3:4, :] += jnp.sum(z2 * z2, axis=0, keepdims=True)

        @pl.when(p == 2)
        def _():
            z2 = z2_scr[rows, :]
            if outer_bn:
                hnew = jnp.maximum(
                    _bn(z2, stats[2:3, :], stats[3:4, :], g2_ref, bt2_ref),
                    0.0)
            else:
                hnew = jnp.maximum(z2, 0.0)
            h_out_ref[...] = hnew
            score_out_ref[...] = (
                score_ref[...]
                + jnp.dot(hnew, pw_ref[...],
                          preferred_element_type=jnp.float32) + pb_ref[...])

    return body


def _full(shape):
    return pl.BlockSpec(shape, lambda p, i: (0, 0))


def _phase_rows(ph, w):
    return pl.BlockSpec(
        (R, w), lambda p, i, _ph=ph: (jnp.where(p == _ph, i, 0), 0))


_LAYER_SPECS = dict(
    grid=(3, G),
    in_specs=[
        _full((1, 1)),
        pl.BlockSpec((NC, R, D),
                     lambda p, i: (0, jnp.where(p == 0, i, 0), 0)),
        _phase_rows(0, D),
        _full((D, D)), _full((1, D)),
        _full((1, D)), _full((1, D)),
        _full((D, D)), _full((1, D)),
        _full((1, D)), _full((1, D)),
        _full((D, OUT)), _full((1, OUT)),
        _phase_rows(2, OUT),
    ],
    out_specs=(_phase_rows(2, D), _phase_rows(2, OUT)),
    out_shape=(jax.ShapeDtypeStruct((N, D), _F32),
               jax.ShapeDtypeStruct((N, OUT), _F32)),
    scratch_shapes=[pltpu.VMEM((N, D), _F32),
                    pltpu.VMEM((N, D), _F32),
                    pltpu.VMEM((8, D), _F32)],
)

def _layer_call(outer_bn):
    return pl.pallas_call(_make_layer_body(outer_bn), **_LAYER_SPECS)


def kernel(batch_features, batch_graphs, mlp_W1, mlp_b1, bn_in_gamma,
           bn_in_beta, mlp_W2, mlp_b2, outer_gamma, outer_beta, pred_W,
           pred_b, eps):
    src = batch_graphs[0]
    dst = batch_graphs[1]
    pad = E_PAD - E
    src_p = jnp.concatenate(
        [src, (jnp.arange(pad, dtype=jnp.int32) * 37) % N])
    dst_p = jnp.concatenate(
        [dst, N + (jnp.arange(pad, dtype=jnp.int32) % 96)])
    src_r = src_p.reshape(NW, NB // CH, CH, EB)
    dst_r = dst_p.reshape(NW, NB // CH, CH, EB)
    zeros = jnp.zeros((SH, D), _F32)

    h = batch_features
    scale = (1.0 + eps).reshape(L - 1, 1, 1)
    score = _score_init(h, pred_W[0], pred_b[0].reshape(1, OUT))
    unit = jnp.ones((1, H), _F32)
    zero_row = jnp.zeros((1, H), _F32)
    for i in range(L - 1):
        parts = _sc_agg(h, src_r, dst_r, zeros)
        if i == 0:
            g2, bt2 = unit, zero_row
            layer = _layer_call(False)
        else:
            g2 = outer_gamma[i - 1].reshape(1, H)
            bt2 = outer_beta[i - 1].reshape(1, H)
            layer = _layer_call(True)
        h, score = layer(
            scale[i], parts, h, mlp_W1[i], mlp_b1[i].reshape(1, H),
            bn_in_gamma[i].reshape(1, H), bn_in_beta[i].reshape(1, H),
            mlp_W2[i], mlp_b2[i].reshape(1, H), g2, bt2, pred_W[i + 1],
            pred_b[i + 1].reshape(1, OUT), score)
    return score


# speedup vs baseline: 1.0682x; 1.0211x over previous
import functools

import jax
import jax.numpy as jnp
from jax import lax
from jax.experimental import pallas as pl
from jax.experimental.pallas import tpu as pltpu
from jax.experimental.pallas import tpu_sc as plsc

N = 10000
E = 320000
D = 128
H = 128
OUT = 64
L = 4

NC = 2
NS = 16
NW = NC * NS
EB = 128
CH = 16
NB = -(-E // (NW * EB * CH)) * CH
E_PAD = NW * NB * EB
SH = (-(-(N + 16) // NS) + 7) // 8 * 8
NP = NS * SH
KB = 2

@functools.lru_cache(maxsize=1)
def _make_sc_agg():
    mesh = plsc.VectorSubcoreMesh(
        core_axis_name="c", subcore_axis_name="s",
        num_cores=NC, num_subcores=NS)

    @functools.partial(
        pl.kernel,
        out_type=jax.ShapeDtypeStruct((NC, NP, D), jnp.float32),
        mesh=mesh,
        scratch_types=[
            pltpu.VMEM((CH, EB), jnp.int32),
            pltpu.VMEM((CH, EB), jnp.int32),
            [pltpu.VMEM((EB, D), jnp.float32) for _ in range(KB)],
            pltpu.VMEM_SHARED((NP, D), jnp.float32),
            [pltpu.SemaphoreType.DMA for _ in range(KB)],
        ],
    )
    def sc_agg(h_hbm, src_hbm, dst_hbm, z_hbm, out_hbm,
               src_v, dst_v, bufs, acc_sh, sems):
        cid = lax.axis_index("c")
        sid = lax.axis_index("s")
        wid = sid * NC + cid
        pltpu.sync_copy(z_hbm, acc_sh.at[pl.ds(sid * SH, SH)])
        plsc.subcore_barrier()

        def body(t, carry):
            pltpu.sync_copy(src_hbm.at[wid, t], src_v)
            pltpu.sync_copy(dst_hbm.at[wid, t], dst_v)
            pltpu.async_copy(h_hbm.at[src_v.at[0]], bufs[0], sems[0])
            for i in range(CH):
                pltpu.make_async_copy(
                    h_hbm.at[pl.ds(0, EB)], bufs[i % KB],
                    sems[i % KB]).wait()
                if i + 1 < CH:
                    pltpu.async_copy(h_hbm.at[src_v.at[i + 1]],
                                     bufs[(i + 1) % KB], sems[(i + 1) % KB])
                pltpu.sync_copy(bufs[i % KB], acc_sh.at[dst_v.at[i]],
                                add=True)
            return carry

        lax.fori_loop(0, NB // CH, body, 0)
        plsc.subcore_barrier()
        pltpu.sync_copy(acc_sh.at[pl.ds(sid * SH, SH)],
                        out_hbm.at[cid, pl.ds(sid * SH, SH)])

    return sc_agg


def _sc_agg(h, src_r, dst_r, zeros):
    return _make_sc_agg()(h, src_r, dst_r, zeros)


R = 2000
G = N // R

_F32 = jnp.float32


def _score_init_body(h_ref, w_ref, b_ref, out_ref):
    out_ref[...] = (
        jnp.dot(h_ref[...], w_ref[...], preferred_element_type=jnp.float32)
        + b_ref[...])


def _score_init(h, w, b):
    return pl.pallas_call(
        _score_init_body,
        grid=(G,),
        in_specs=[pl.BlockSpec((R, D), lambda i: (i, 0)),
                  pl.BlockSpec((D, OUT), lambda i: (0, 0)),
                  pl.BlockSpec((1, OUT), lambda i: (0, 0))],
        out_specs=pl.BlockSpec((R, OUT), lambda i: (i, 0)),
        out_shape=jax.ShapeDtypeStruct((N, OUT), _F32),
    )(h, w, b)


def _bn(x, mu_row, sq_row, g_ref, bt_ref):
    mu = mu_row * (1.0 / N)
    var = sq_row * (1.0 / N) - mu * mu
    inv = g_ref[...] * lax.rsqrt(var + 1e-5)
    return (x - mu) * inv + bt_ref[...]


def _make_layer_body(outer_bn):
    def body(scale_ref, agg_ref, h_ref, w1_ref, b1_ref, g1_ref, bt1_ref,
             w2_ref, b2_ref, g2_ref, bt2_ref, pw_ref, pb_ref, score_ref,
             h_out_ref, score_out_ref, z1_scr, z2_scr, stats):
        p = pl.program_id(0)
        i = pl.program_id(1)
        rows = pl.ds(i * R, R)

        @pl.when(p == 0)
        def _():
            @pl.when(i == 0)
            def _():
                stats[...] = jnp.zeros_like(stats)
            pooled = agg_ref[0] + agg_ref[1] + scale_ref[...] * h_ref[...]
            z = (jnp.dot(pooled, w1_ref[...],
                         preferred_element_type=jnp.float32) + b1_ref[...])
            z1_scr[rows, :] = z
            stats[0:1, :] += jnp.sum(z, axis=0, keepdims=True)
            stats[1:2, :] += jnp.sum(z * z, axis=0, keepdims=True)

        @pl.when(p == 1)
        def _():
            z1 = z1_scr[rows, :]
            a = jnp.maximum(
                _bn(z1, stats[0:1, :], stats[1:2, :], g1_ref, bt1_ref), 0.0)
            z2 = (jnp.dot(a, w2_ref[...],
                          preferred_element_type=jnp.float32) + b2_ref[...])
            z2_scr[rows, :] = z2
            stats[2:3, :] += jnp.sum(z2, axis=0, keepdims=True)
            stats[3:4, :] += jnp.sum(z2 * z2, axis=0, keepdims=True)

        @pl.when(p == 2)
        def _():
            z2 = z2_scr[rows, :]
            if outer_bn:
                hnew = jnp.maximum(
                    _bn(z2, stats[2:3, :], stats[3:4, :], g2_ref, bt2_ref),
                    0.0)
            else:
                hnew = jnp.maximum(z2, 0.0)
            h_out_ref[...] = hnew
            score_out_ref[...] = (
                score_ref[...]
                + jnp.dot(hnew, pw_ref[...],
                          preferred_element_type=jnp.float32) + pb_ref[...])

    return body


def _full(shape):
    return pl.BlockSpec(shape, lambda p, i: (0, 0))


def _phase_rows(ph, w):
    return pl.BlockSpec(
        (R, w), lambda p, i, _ph=ph: (jnp.where(p == _ph, i, 0), 0))


_LAYER_SPECS = dict(
    grid=(3, G),
    in_specs=[
        _full((1, 1)),
        pl.BlockSpec((NC, R, D),
                     lambda p, i: (0, jnp.where(p == 0, i, 0), 0)),
        _phase_rows(0, D),
        _full((D, D)), _full((1, D)),
        _full((1, D)), _full((1, D)),
        _full((D, D)), _full((1, D)),
        _full((1, D)), _full((1, D)),
        _full((D, OUT)), _full((1, OUT)),
        _phase_rows(2, OUT),
    ],
    out_specs=(_phase_rows(2, D), _phase_rows(2, OUT)),
    out_shape=(jax.ShapeDtypeStruct((N, D), _F32),
               jax.ShapeDtypeStruct((N, OUT), _F32)),
    scratch_shapes=[pltpu.VMEM((N, D), _F32),
                    pltpu.VMEM((N, D), _F32),
                    pltpu.VMEM((8, D), _F32)],
)

def _layer_call(outer_bn):
    return pl.pallas_call(_make_layer_body(outer_bn), **_LAYER_SPECS)


def kernel(batch_features, batch_graphs, mlp_W1, mlp_b1, bn_in_gamma,
           bn_in_beta, mlp_W2, mlp_b2, outer_gamma, outer_beta, pred_W,
           pred_b, eps):
    src = batch_graphs[0]
    dst = batch_graphs[1]
    pad = E_PAD - E
    src_p = jnp.concatenate(
        [src, (jnp.arange(pad, dtype=jnp.int32) * 37) % N])
    dst_p = jnp.concatenate(
        [dst, N + (jnp.arange(pad, dtype=jnp.int32) % 96)])
    src_r = src_p.reshape(NW, NB // CH, CH, EB)
    dst_r = dst_p.reshape(NW, NB // CH, CH, EB)
    zeros = jnp.zeros((SH, D), _F32)

    h = batch_features
    scale = (1.0 + eps).reshape(L - 1, 1, 1)
    score = _score_init(h, pred_W[0], pred_b[0].reshape(1, OUT))
    unit = jnp.ones((1, H), _F32)
    zero_row = jnp.zeros((1, H), _F32)
    for i in range(L - 1):
        parts = _sc_agg(h, src_r, dst_r, zeros)
        if i == 0:
            g2, bt2 = unit, zero_row
            layer = _layer_call(False)
        else:
            g2 = outer_gamma[i - 1].reshape(1, H)
            bt2 = outer_beta[i - 1].reshape(1, H)
            layer = _layer_call(True)
        h, score = layer(
            scale[i], parts, h, mlp_W1[i], mlp_b1[i].reshape(1, H),
            bn_in_gamma[i].reshape(1, H), bn_in_beta[i].reshape(1, H),
            mlp_W2[i], mlp_b2[i].reshape(1, H), g2, bt2, pred_W[i + 1],
            pred_b[i + 1].reshape(1, OUT), score)
    return score


# speedup vs baseline: 1.0815x; 1.0125x over previous
import functools

import jax
import jax.numpy as jnp
from jax import lax
from jax.experimental import pallas as pl
from jax.experimental.pallas import tpu as pltpu
from jax.experimental.pallas import tpu_sc as plsc

N = 10000
E = 320000
D = 128
H = 128
OUT = 64
L = 4

NC = 2
NS = 16
NW = NC * NS
EB = 128
CH = 20
NB = -(-E // (NW * EB * CH)) * CH
E_PAD = NW * NB * EB
SH = (-(-(N + 16) // NS) + 7) // 8 * 8
NP = NS * SH
KB = 2

@functools.lru_cache(maxsize=1)
def _make_sc_agg():
    mesh = plsc.VectorSubcoreMesh(
        core_axis_name="c", subcore_axis_name="s",
        num_cores=NC, num_subcores=NS)

    @functools.partial(
        pl.kernel,
        out_type=jax.ShapeDtypeStruct((NC, NP, D), jnp.float32),
        mesh=mesh,
        scratch_types=[
            pltpu.VMEM((CH, EB), jnp.int32),
            pltpu.VMEM((CH, EB), jnp.int32),
            [pltpu.VMEM((EB, D), jnp.float32) for _ in range(KB)],
            pltpu.VMEM_SHARED((NP, D), jnp.float32),
            [pltpu.SemaphoreType.DMA for _ in range(KB)],
        ],
    )
    def sc_agg(h_hbm, src_hbm, dst_hbm, z_hbm, out_hbm,
               src_v, dst_v, bufs, acc_sh, sems):
        cid = lax.axis_index("c")
        sid = lax.axis_index("s")
        wid = sid * NC + cid
        pltpu.sync_copy(z_hbm, acc_sh.at[pl.ds(sid * SH, SH)])
        plsc.subcore_barrier()

        def body(t, carry):
            pltpu.sync_copy(src_hbm.at[wid, t], src_v)
            pltpu.sync_copy(dst_hbm.at[wid, t], dst_v)
            pltpu.async_copy(h_hbm.at[src_v.at[0]], bufs[0], sems[0])
            for i in range(CH):
                pltpu.make_async_copy(
                    h_hbm.at[pl.ds(0, EB)], bufs[i % KB],
                    sems[i % KB]).wait()
                if i + 1 < CH:
                    pltpu.async_copy(h_hbm.at[src_v.at[i + 1]],
                                     bufs[(i + 1) % KB], sems[(i + 1) % KB])
                pltpu.sync_copy(bufs[i % KB], acc_sh.at[dst_v.at[i]],
                                add=True)
            return carry

        lax.fori_loop(0, NB // CH, body, 0)
        plsc.subcore_barrier()
        pltpu.sync_copy(acc_sh.at[pl.ds(sid * SH, SH)],
                        out_hbm.at[cid, pl.ds(sid * SH, SH)])

    return sc_agg


def _sc_agg(h, src_r, dst_r, zeros):
    return _make_sc_agg()(h, src_r, dst_r, zeros)


R = 2000
G = N // R

_F32 = jnp.float32


def _score_init_body(h_ref, w_ref, b_ref, out_ref):
    out_ref[...] = (
        jnp.dot(h_ref[...], w_ref[...], preferred_element_type=jnp.float32)
        + b_ref[...])


def _score_init(h, w, b):
    return pl.pallas_call(
        _score_init_body,
        grid=(G,),
        in_specs=[pl.BlockSpec((R, D), lambda i: (i, 0)),
                  pl.BlockSpec((D, OUT), lambda i: (0, 0)),
                  pl.BlockSpec((1, OUT), lambda i: (0, 0))],
        out_specs=pl.BlockSpec((R, OUT), lambda i: (i, 0)),
        out_shape=jax.ShapeDtypeStruct((N, OUT), _F32),
    )(h, w, b)


def _bn(x, mu_row, sq_row, g_ref, bt_ref):
    mu = mu_row * (1.0 / N)
    var = sq_row * (1.0 / N) - mu * mu
    inv = g_ref[...] * lax.rsqrt(var + 1e-5)
    return (x - mu) * inv + bt_ref[...]


def _make_layer_body(outer_bn):
    def body(scale_ref, agg_ref, h_ref, w1_ref, b1_ref, g1_ref, bt1_ref,
             w2_ref, b2_ref, g2_ref, bt2_ref, pw_ref, pb_ref, score_ref,
             h_out_ref, score_out_ref, z1_scr, z2_scr, stats):
        p = pl.program_id(0)
        i = pl.program_id(1)
        rows = pl.ds(i * R, R)

        @pl.when(p == 0)
        def _():
            @pl.when(i == 0)
            def _():
                stats[...] = jnp.zeros_like(stats)
            pooled = agg_ref[0] + agg_ref[1] + scale_ref[...] * h_ref[...]
            z = (jnp.dot(pooled, w1_ref[...],
                         preferred_element_type=jnp.float32) + b1_ref[...])
            z1_scr[rows, :] = z
            stats[0:1, :] += jnp.sum(z, axis=0, keepdims=True)
            stats[1:2, :] += jnp.sum(z * z, axis=0, keepdims=True)

        @pl.when(p == 1)
        def _():
            z1 = z1_scr[rows, :]
            a = jnp.maximum(
                _bn(z1, stats[0:1, :], stats[1:2, :], g1_ref, bt1_ref), 0.0)
            z2 = (jnp.dot(a, w2_ref[...],
                          preferred_element_type=jnp.float32) + b2_ref[...])
            z2_scr[rows, :] = z2
            stats[2:3, :] += jnp.sum(z2, axis=0, keepdims=True)
            stats[3:4, :] += jnp.sum(z2 * z2, axis=0, keepdims=True)

        @pl.when(p == 2)
        def _():
            z2 = z2_scr[rows, :]
            if outer_bn:
                hnew = jnp.maximum(
                    _bn(z2, stats[2:3, :], stats[3:4, :], g2_ref, bt2_ref),
                    0.0)
            else:
                hnew = jnp.maximum(z2, 0.0)
            h_out_ref[...] = hnew
            score_out_ref[...] = (
                score_ref[...]
                + jnp.dot(hnew, pw_ref[...],
                          preferred_element_type=jnp.float32) + pb_ref[...])

    return body


def _full(shape):
    return pl.BlockSpec(shape, lambda p, i: (0, 0))


def _phase_rows(ph, w):
    return pl.BlockSpec(
        (R, w), lambda p, i, _ph=ph: (jnp.where(p == _ph, i, 0), 0))


_LAYER_SPECS = dict(
    grid=(3, G),
    in_specs=[
        _full((1, 1)),
        pl.BlockSpec((NC, R, D),
                     lambda p, i: (0, jnp.where(p == 0, i, 0), 0)),
        _phase_rows(0, D),
        _full((D, D)), _full((1, D)),
        _full((1, D)), _full((1, D)),
        _full((D, D)), _full((1, D)),
        _full((1, D)), _full((1, D)),
        _full((D, OUT)), _full((1, OUT)),
        _phase_rows(2, OUT),
    ],
    out_specs=(_phase_rows(2, D), _phase_rows(2, OUT)),
    out_shape=(jax.ShapeDtypeStruct((N, D), _F32),
               jax.ShapeDtypeStruct((N, OUT), _F32)),
    scratch_shapes=[pltpu.VMEM((N, D), _F32),
                    pltpu.VMEM((N, D), _F32),
                    pltpu.VMEM((8, D), _F32)],
)

def _layer_call(outer_bn):
    return pl.pallas_call(_make_layer_body(outer_bn), **_LAYER_SPECS)


def kernel(batch_features, batch_graphs, mlp_W1, mlp_b1, bn_in_gamma,
           bn_in_beta, mlp_W2, mlp_b2, outer_gamma, outer_beta, pred_W,
           pred_b, eps):
    src = batch_graphs[0]
    dst = batch_graphs[1]
    pad = E_PAD - E
    src_p = jnp.concatenate(
        [src, (jnp.arange(pad, dtype=jnp.int32) * 37) % N])
    dst_p = jnp.concatenate(
        [dst, N + (jnp.arange(pad, dtype=jnp.int32) % 96)])
    src_r = src_p.reshape(NW, NB // CH, CH, EB)
    dst_r = dst_p.reshape(NW, NB // CH, CH, EB)
    zeros = jnp.zeros((SH, D), _F32)

    h = batch_features
    scale = (1.0 + eps).reshape(L - 1, 1, 1)
    score = _score_init(h, pred_W[0], pred_b[0].reshape(1, OUT))
    unit = jnp.ones((1, H), _F32)
    zero_row = jnp.zeros((1, H), _F32)
    for i in range(L - 1):
        parts = _sc_agg(h, src_r, dst_r, zeros)
        if i == 0:
            g2, bt2 = unit, zero_row
            layer = _layer_call(False)
        else:
            g2 = outer_gamma[i - 1].reshape(1, H)
            bt2 = outer_beta[i - 1].reshape(1, H)
            layer = _layer_call(True)
        h, score = layer(
            scale[i], parts, h, mlp_W1[i], mlp_b1[i].reshape(1, H),
            bn_in_gamma[i].reshape(1, H), bn_in_beta[i].reshape(1, H),
            mlp_W2[i], mlp_b2[i].reshape(1, H), g2, bt2, pred_W[i + 1],
            pred_b[i + 1].reshape(1, OUT), score)
    return score


# speedup vs baseline: 1.1024x; 1.0193x over previous
import functools

import jax
import jax.numpy as jnp
from jax import lax
from jax.experimental import pallas as pl
from jax.experimental.pallas import tpu as pltpu
from jax.experimental.pallas import tpu_sc as plsc

N = 10000
E = 320000
D = 128
H = 128
OUT = 64
L = 4

NC = 2
NS = 16
NW = NC * NS
EB = 128
CH = 40
NB = -(-E // (NW * EB * CH)) * CH
E_PAD = NW * NB * EB
SH = (-(-(N + 16) // NS) + 7) // 8 * 8
NP = NS * SH
KB = 2

@functools.lru_cache(maxsize=1)
def _make_sc_agg():
    mesh = plsc.VectorSubcoreMesh(
        core_axis_name="c", subcore_axis_name="s",
        num_cores=NC, num_subcores=NS)

    @functools.partial(
        pl.kernel,
        out_type=jax.ShapeDtypeStruct((NC, NP, D), jnp.float32),
        mesh=mesh,
        scratch_types=[
            pltpu.VMEM((CH, EB), jnp.int32),
            pltpu.VMEM((CH, EB), jnp.int32),
            [pltpu.VMEM((EB, D), jnp.float32) for _ in range(KB)],
            pltpu.VMEM_SHARED((NP, D), jnp.float32),
            [pltpu.SemaphoreType.DMA for _ in range(KB)],
        ],
    )
    def sc_agg(h_hbm, src_hbm, dst_hbm, z_hbm, out_hbm,
               src_v, dst_v, bufs, acc_sh, sems):
        cid = lax.axis_index("c")
        sid = lax.axis_index("s")
        wid = sid * NC + cid
        pltpu.sync_copy(z_hbm, acc_sh.at[pl.ds(sid * SH, SH)])
        plsc.subcore_barrier()

        def body(t, carry):
            pltpu.sync_copy(src_hbm.at[wid, t], src_v)
            pltpu.sync_copy(dst_hbm.at[wid, t], dst_v)
            pltpu.async_copy(h_hbm.at[src_v.at[0]], bufs[0], sems[0])
            for i in range(CH):
                pltpu.make_async_copy(
                    h_hbm.at[pl.ds(0, EB)], bufs[i % KB],
                    sems[i % KB]).wait()
                if i + 1 < CH:
                    pltpu.async_copy(h_hbm.at[src_v.at[i + 1]],
                                     bufs[(i + 1) % KB], sems[(i + 1) % KB])
                pltpu.sync_copy(bufs[i % KB], acc_sh.at[dst_v.at[i]],
                                add=True)
            return carry

        lax.fori_loop(0, NB // CH, body, 0)
        plsc.subcore_barrier()
        pltpu.sync_copy(acc_sh.at[pl.ds(sid * SH, SH)],
                        out_hbm.at[cid, pl.ds(sid * SH, SH)])

    return sc_agg


def _sc_agg(h, src_r, dst_r, zeros):
    return _make_sc_agg()(h, src_r, dst_r, zeros)


R = 2000
G = N // R

_F32 = jnp.float32


def _score_init_body(h_ref, w_ref, b_ref, out_ref):
    out_ref[...] = (
        jnp.dot(h_ref[...], w_ref[...], preferred_element_type=jnp.float32)
        + b_ref[...])


def _score_init(h, w, b):
    return pl.pallas_call(
        _score_init_body,
        grid=(G,),
        in_specs=[pl.BlockSpec((R, D), lambda i: (i, 0)),
                  pl.BlockSpec((D, OUT), lambda i: (0, 0)),
                  pl.BlockSpec((1, OUT), lambda i: (0, 0))],
        out_specs=pl.BlockSpec((R, OUT), lambda i: (i, 0)),
        out_shape=jax.ShapeDtypeStruct((N, OUT), _F32),
    )(h, w, b)


def _bn(x, mu_row, sq_row, g_ref, bt_ref):
    mu = mu_row * (1.0 / N)
    var = sq_row * (1.0 / N) - mu * mu
    inv = g_ref[...] * lax.rsqrt(var + 1e-5)
    return (x - mu) * inv + bt_ref[...]


def _make_layer_body(outer_bn):
    def body(scale_ref, agg_ref, h_ref, w1_ref, b1_ref, g1_ref, bt1_ref,
             w2_ref, b2_ref, g2_ref, bt2_ref, pw_ref, pb_ref, score_ref,
             h_out_ref, score_out_ref, z1_scr, z2_scr, stats):
        p = pl.program_id(0)
        i = pl.program_id(1)
        rows = pl.ds(i * R, R)

        @pl.when(p == 0)
        def _():
            @pl.when(i == 0)
            def _():
                stats[...] = jnp.zeros_like(stats)
            pooled = agg_ref[0] + agg_ref[1] + scale_ref[...] * h_ref[...]
            z = (jnp.dot(pooled, w1_ref[...],
                         preferred_element_type=jnp.float32) + b1_ref[...])
            z1_scr[rows, :] = z
            stats[0:1, :] += jnp.sum(z, axis=0, keepdims=True)
            stats[1:2, :] += jnp.sum(z * z, axis=0, keepdims=True)

        @pl.when(p == 1)
        def _():
            z1 = z1_scr[rows, :]
            a = jnp.maximum(
                _bn(z1, stats[0:1, :], stats[1:2, :], g1_ref, bt1_ref), 0.0)
            z2 = (jnp.dot(a, w2_ref[...],
                          preferred_element_type=jnp.float32) + b2_ref[...])
            z2_scr[rows, :] = z2
            stats[2:3, :] += jnp.sum(z2, axis=0, keepdims=True)
            stats[3:4, :] += jnp.sum(z2 * z2, axis=0, keepdims=True)

        @pl.when(p == 2)
        def _():
            z2 = z2_scr[rows, :]
            if outer_bn:
                hnew = jnp.maximum(
                    _bn(z2, stats[2:3, :], stats[3:4, :], g2_ref, bt2_ref),
                    0.0)
            else:
                hnew = jnp.maximum(z2, 0.0)
            h_out_ref[...] = hnew
            score_out_ref[...] = (
                score_ref[...]
                + jnp.dot(hnew, pw_ref[...],
                          preferred_element_type=jnp.float32) + pb_ref[...])

    return body


def _full(shape):
    return pl.BlockSpec(shape, lambda p, i: (0, 0))


def _phase_rows(ph, w):
    return pl.BlockSpec(
        (R, w), lambda p, i, _ph=ph: (jnp.where(p == _ph, i, 0), 0))


_LAYER_SPECS = dict(
    grid=(3, G),
    in_specs=[
        _full((1, 1)),
        pl.BlockSpec((NC, R, D),
                     lambda p, i: (0, jnp.where(p == 0, i, 0), 0)),
        _phase_rows(0, D),
        _full((D, D)), _full((1, D)),
        _full((1, D)), _full((1, D)),
        _full((D, D)), _full((1, D)),
        _full((1, D)), _full((1, D)),
        _full((D, OUT)), _full((1, OUT)),
        _phase_rows(2, OUT),
    ],
    out_specs=(_phase_rows(2, D), _phase_rows(2, OUT)),
    out_shape=(jax.ShapeDtypeStruct((N, D), _F32),
               jax.ShapeDtypeStruct((N, OUT), _F32)),
    scratch_shapes=[pltpu.VMEM((N, D), _F32),
                    pltpu.VMEM((N, D), _F32),
                    pltpu.VMEM((8, D), _F32)],
)

def _layer_call(outer_bn):
    return pl.pallas_call(_make_layer_body(outer_bn), **_LAYER_SPECS)


def kernel(batch_features, batch_graphs, mlp_W1, mlp_b1, bn_in_gamma,
           bn_in_beta, mlp_W2, mlp_b2, outer_gamma, outer_beta, pred_W,
           pred_b, eps):
    src = batch_graphs[0]
    dst = batch_graphs[1]
    pad = E_PAD - E
    src_p = jnp.concatenate(
        [src, (jnp.arange(pad, dtype=jnp.int32) * 37) % N])
    dst_p = jnp.concatenate(
        [dst, N + (jnp.arange(pad, dtype=jnp.int32) % 96)])
    src_r = src_p.reshape(NW, NB // CH, CH, EB)
    dst_r = dst_p.reshape(NW, NB // CH, CH, EB)
    zeros = jnp.zeros((SH, D), _F32)

    h = batch_features
    scale = (1.0 + eps).reshape(L - 1, 1, 1)
    score = _score_init(h, pred_W[0], pred_b[0].reshape(1, OUT))
    unit = jnp.ones((1, H), _F32)
    zero_row = jnp.zeros((1, H), _F32)
    for i in range(L - 1):
        parts = _sc_agg(h, src_r, dst_r, zeros)
        if i == 0:
            g2, bt2 = unit, zero_row
            layer = _layer_call(False)
        else:
            g2 = outer_gamma[i - 1].reshape(1, H)
            bt2 = outer_beta[i - 1].reshape(1, H)
            layer = _layer_call(True)
        h, score = layer(
            scale[i], parts, h, mlp_W1[i], mlp_b1[i].reshape(1, H),
            bn_in_gamma[i].reshape(1, H), bn_in_beta[i].reshape(1, H),
            mlp_W2[i], mlp_b2[i].reshape(1, H), g2, bt2, pred_W[i + 1],
            pred_b[i + 1].reshape(1, OUT), score)
    return score
